# Initial kernel scaffold; baseline (speedup 1.0000x reference)
import jax
import jax.numpy as jnp
from jax.experimental import pallas as pl


def kernel(h_a, x_a, e_a_idx, e_a_type, e_a_attr, coord_diff_a, h_f, x_f, bm_mat, bond_emb, W1a, b1a, W2a, b2a, W3a, W1f, b1f, W2f, b2f, W3f):
    raise NotImplementedError("write your pallas kernel here")


# speedup vs baseline: 2.4081x; 2.4081x over previous
import functools

import jax
import jax.numpy as jnp
from jax import lax
from jax.experimental import pallas as pl
from jax.experimental.pallas import tpu as pltpu
from jax.experimental.pallas import tpu_sc as plsc

N = 10000
E = 320000
H = 128
F = 512
NORM_FACTOR = 100.0

NP = 10240
EP = 327680
TILES = 32
CPT = EP // TILES
CHUNK = 128
NCH = CPT // CHUNK
BN = 2048
BE = 2048

_F32 = jnp.float32
_BF16 = jnp.bfloat16


def _silu(x):
    return x / (1.0 + jnp.exp(-x))


def _silu_bf(x):
    xb = x.astype(_BF16)
    one = jnp.ones((), _BF16)
    return xb / (one + jnp.exp(-xb))


def _bf(x):
    return x.astype(_BF16)


def _dense_body(h_ref, x4_ref, bm_ref, hf_ref, xf4_ref, bond_ref,
                wr_ref, wc_ref, wbe_ref, b1a_ref,
                w1fh_ref, w1fb_ref, wrad_ref, b1f_ref, w2f_ref, b2f_ref,
                w3f_ref,
                pr_ref, pc_ref, tbe_ref, part_ref):
    h = h_ref[...]
    hb = _bf(h)
    pr_ref[...] = jnp.dot(hb, _bf(wr_ref[...]), preferred_element_type=_F32)
    pc_ref[...] = jnp.dot(hb, _bf(wc_ref[...]), preferred_element_type=_F32)

    bm = bm_ref[...]
    bmh = jnp.dot(_bf(bm), _bf(hf_ref[...]), preferred_element_type=_F32)
    mx = jnp.dot(bm, xf4_ref[...], preferred_element_type=_F32)
    x4 = x4_ref[...]
    cdf = x4 - mx
    radial = jnp.sum(cdf * cdf, axis=1, keepdims=True)
    norm = jnp.sqrt(radial + 1e-8)
    cdfn = cdf / (norm + 1.0)

    z1 = (jnp.dot(hb, _bf(w1fh_ref[...]), preferred_element_type=_F32)
          + jnp.dot(_bf(bmh), _bf(w1fb_ref[...]), preferred_element_type=_F32)
          + radial * wrad_ref[0:1, :] + b1f_ref[0:1, :])
    h1 = _silu(z1)
    h2 = _silu(jnp.dot(_bf(h1), _bf(w2f_ref[...]), preferred_element_type=_F32)
               + b2f_ref[0:1, :])
    sf = jnp.sum(h2 * w3f_ref[0:1, :], axis=1, keepdims=True)
    part_ref[...] = x4 + cdfn * sf

    @pl.when(pl.program_id(0) == 0)
    def _():
        tbe_ref[...] = (jnp.dot(_bf(bond_ref[...]), _bf(wbe_ref[...]),
                                preferred_element_type=_F32)
                        + b1a_ref[0:1, :])


def _dense_call(h_p, x4, bm_p, h_f, xf4, bond_p, Wr, Wc, Wbe, b1a2,
                W1fh, W1fb, wrad, b1f2, W2f, b2f2, w3f2):
    nblk = NP // BN
    full = lambda s: pl.BlockSpec(s, lambda i: tuple(0 for _ in s))
    return pl.pallas_call(
        _dense_body,
        grid=(nblk,),
        in_specs=[
            pl.BlockSpec((BN, H), lambda i: (i, 0)),
            pl.BlockSpec((BN, 4), lambda i: (i, 0)),
            pl.BlockSpec((BN, F), lambda i: (i, 0)),
            full((F, H)),
            full((F, 4)),
            full((128, H)),
            full((H, H)), full((H, H)), full((H, H)),
            full((8, H)),
            full((H, H)), full((H, H)),
            full((8, H)), full((8, H)),
            full((H, H)), full((8, H)), full((8, H)),
        ],
        out_specs=[
            pl.BlockSpec((BN, H), lambda i: (i, 0)),
            pl.BlockSpec((BN, H), lambda i: (i, 0)),
            pl.BlockSpec((128, H), lambda i: (0, 0)),
            pl.BlockSpec((BN, 4), lambda i: (i, 0)),
        ],
        out_shape=[
            jax.ShapeDtypeStruct((NP, H), _F32),
            jax.ShapeDtypeStruct((NP, H), _F32),
            jax.ShapeDtypeStruct((128, H), _F32),
            jax.ShapeDtypeStruct((NP, 4), _F32),
        ],
    )(h_p, x4, bm_p, h_f, xf4, bond_p, Wr, Wc, Wbe, b1a2,
      W1fh, W1fb, wrad, b1f2, W2f, b2f2, w3f2)


def _sc_gather(prow, pcol, row_sc, col_sc):
    mesh = plsc.VectorSubcoreMesh(core_axis_name="c", subcore_axis_name="s")

    @functools.partial(
        pl.kernel, mesh=mesh,
        out_type=[jax.ShapeDtypeStruct((EP, H), _F32),
                  jax.ShapeDtypeStruct((EP, H), _F32)],
        scratch_types=[
            pltpu.VMEM((NCH, CHUNK), jnp.int32),
            pltpu.VMEM((NCH, CHUNK), jnp.int32),
            pltpu.VMEM((CHUNK, H), _F32),
            pltpu.VMEM((CHUNK, H), _F32),
            pltpu.SemaphoreType.DMA,
            pltpu.SemaphoreType.DMA,
        ],
    )
    def gather_k(pr_hbm, pc_hbm, ri_hbm, ci_hbm, gr_hbm, gc_hbm,
                 ir_v, ic_v, bufr, bufc, semr, semc):
        wid = lax.axis_index("s") * 2 + lax.axis_index("c")
        base = wid * CPT
        pltpu.async_copy(ri_hbm.at[wid], ir_v, semr).wait()
        pltpu.async_copy(ci_hbm.at[wid], ic_v, semc).wait()

        @pl.loop(0, NCH)
        def _(j):
            cr = pltpu.async_copy(pr_hbm.at[ir_v.at[j]], bufr, semr)
            cc = pltpu.async_copy(pc_hbm.at[ic_v.at[j]], bufc, semc)
            cr.wait()
            pltpu.async_copy(
                bufr, gr_hbm.at[pl.ds(base + j * CHUNK, CHUNK)], semr).wait()
            cc.wait()
            pltpu.async_copy(
                bufc, gc_hbm.at[pl.ds(base + j * CHUNK, CHUNK)], semc).wait()

    return gather_k(prow, pcol, row_sc, col_sc)


def _edge_body(gr_ref, gc_ref, ty_ref, at_ref, cd_ref,
               tbe_ref, wat_ref, w2a_ref, b2a_ref, w3a_ref, t_ref):
    z = gr_ref[...] + gc_ref[...]
    ids = lax.broadcasted_iota(jnp.int32, (BE, H), 1)
    oh = _bf(ty_ref[...] == ids)
    zbe = jnp.dot(oh, _bf(tbe_ref[...]), preferred_element_type=_F32)
    at = at_ref[...]
    z = z + zbe + at[:, 0:1] * wat_ref[0:1, :] + at[:, 1:2] * wat_ref[1:2, :]
    h1 = _silu(z)
    h2 = _silu(jnp.dot(_bf(h1), _bf(w2a_ref[...]), preferred_element_type=_F32)
               + b2a_ref[0:1, :])
    s = jnp.sum(h2 * w3a_ref[0:1, :], axis=1, keepdims=True)
    t_ref[...] = cd_ref[...] * s


def _edge_call(grow, gcol, ty2, attr_p, cd4, tbe, Wat, W2a, b2a2, w3a2):
    nblk = EP // BE
    full = lambda s: pl.BlockSpec(s, lambda i: tuple(0 for _ in s))
    return pl.pallas_call(
        _edge_body,
        grid=(nblk,),
        in_specs=[
            pl.BlockSpec((BE, H), lambda i: (i, 0)),
            pl.BlockSpec((BE, H), lambda i: (i, 0)),
            pl.BlockSpec((BE, 1), lambda i: (i, 0)),
            pl.BlockSpec((BE, 2), lambda i: (i, 0)),
            pl.BlockSpec((BE, 4), lambda i: (i, 0)),
            full((128, H)), full((8, H)), full((H, H)),
            full((8, H)), full((8, H)),
        ],
        out_specs=pl.BlockSpec((BE, 4), lambda i: (i, 0)),
        out_shape=jax.ShapeDtypeStruct((EP, 4), _F32),
    )(grow, gcol, ty2, attr_p, cd4, tbe, Wat, W2a, b2a2, w3a2)


def _sc_scatter(t_r, row_sc2):
    mesh = plsc.VectorSubcoreMesh(core_axis_name="c", subcore_axis_name="s")
    npass = 4
    gp = CPT // 16 // npass
    accr = NP * 4 // 128

    @functools.partial(
        pl.kernel, mesh=mesh,
        compiler_params=pltpu.CompilerParams(needs_layout_passes=False),
        out_type=jax.ShapeDtypeStruct((TILES, accr, 128), _F32),
        scratch_types=[
            pltpu.VMEM((gp * 16 // 128, 128), jnp.int32),
            pltpu.VMEM((gp * 64 // 128, 128), _F32),
            pltpu.VMEM((accr, 128), _F32),
            pltpu.SemaphoreType.DMA,
        ],
    )
    def scatter_k(t_hbm, ri_hbm, acc_hbm, idx_v, t_v, acc_v, sem):
        wid = lax.axis_index("s") * 2 + lax.axis_index("c")

        zeros = jnp.zeros((16,), _F32)

        @pl.loop(0, accr)
        def _(r):
            @pl.loop(0, 8)
            def _(i):
                acc_v[r, pl.ds(i * 16, 16)] = zeros

        lane4 = lax.iota(jnp.int32, 16) * 4

        @pl.loop(0, npass)
        def _(p):
            pltpu.async_copy(t_hbm.at[wid * npass + p], t_v, sem).wait()
            pltpu.async_copy(ri_hbm.at[wid * npass + p], idx_v, sem).wait()

            @pl.loop(0, gp)
            def _(k):
                r = lax.shift_right_logical(k, 3)
                o = lax.shift_left(jnp.bitwise_and(k, 7), 4)
                idxn = idx_v[r, pl.ds(o, 16)]
                idx4 = lax.shift_left(idxn, 2)
                for c in range(3):
                    w = lane4 + (k * 64 + c)
                    vals = plsc.load_gather(
                        t_v, [lax.shift_right_logical(w, 7),
                              jnp.bitwise_and(w, 127)])
                    idxf = idx4 + c
                    plsc.addupdate_scatter(
                        acc_v, [lax.shift_right_logical(idxf, 7),
                                jnp.bitwise_and(idxf, 127)], vals)

        pltpu.async_copy(acc_v, acc_hbm.at[wid], sem).wait()

    return scatter_k(t_r, row_sc2)


def _final_body(acc_ref, part_ref, out_ref):
    agg = jnp.sum(acc_ref[...], axis=0) * (1.0 / NORM_FACTOR)
    out_ref[...] = part_ref[...] + agg


def _final_call(acc_r, part_r):
    return pl.pallas_call(
        _final_body,
        out_shape=jax.ShapeDtypeStruct((8, NP * 4 // 8), _F32),
    )(acc_r, part_r)


def kernel(h_a, x_a, e_a_idx, e_a_type, e_a_attr, coord_diff_a, h_f, x_f,
           bm_mat, bond_emb, W1a, b1a, W2a, b2a, W3a,
           W1f, b1f, W2f, b2f, W3f):
    padN = NP - N
    padE = EP - E

    h_p = jnp.pad(h_a, ((0, padN), (0, 0)))
    x4 = jnp.pad(x_a, ((0, padN), (0, 1)))
    bm_p = jnp.pad(bm_mat, ((0, padN), (0, 0)))
    xf4 = jnp.pad(x_f, ((0, 0), (0, 1)))
    bond_p = jnp.pad(bond_emb, ((0, 28), (0, 0)))

    row = e_a_idx[0]
    col = e_a_idx[1]
    ipad = jnp.full((padE,), N, jnp.int32)
    row_p = jnp.concatenate([row, ipad])
    col_p = jnp.concatenate([col, ipad])
    ty2 = jnp.pad(e_a_type, (0, padE)).reshape(EP, 1)
    attr_p = jnp.pad(e_a_attr, ((0, padE), (0, 0)))
    cd4 = jnp.pad(coord_diff_a, ((0, padE), (0, 1)))

    pad8 = lambda v: jnp.pad(v.reshape(1, H), ((0, 7), (0, 0)))
    Wr = W1a[0:H]
    Wc = W1a[H:2 * H]
    Wat = jnp.pad(W1a[2 * H:2 * H + 2], ((0, 6), (0, 0)))
    Wbe = W1a[2 * H + 2:]
    b1a2 = pad8(b1a)
    b2a2 = pad8(b2a)
    w3a2 = pad8(W3a.reshape(H))
    W1fh = W1f[0:H]
    W1fb = W1f[H:2 * H]
    wrad = pad8(W1f[2 * H] + W1f[2 * H + 1])
    b1f2 = pad8(b1f)
    b2f2 = pad8(b2f)
    w3f2 = pad8(W3f.reshape(H))

    prow, pcol, tbe, part = _dense_call(
        h_p, x4, bm_p, h_f, xf4, bond_p, Wr, Wc, Wbe, b1a2,
        W1fh, W1fb, wrad, b1f2, W2f, b2f2, w3f2)

    row_sc = row_p.reshape(TILES, NCH, CHUNK)
    col_sc = col_p.reshape(TILES, NCH, CHUNK)
    grow, gcol = _sc_gather(prow, pcol, row_sc, col_sc)

    t4 = _edge_call(grow, gcol, ty2, attr_p, cd4, tbe, Wat, W2a, b2a2, w3a2)

    t_r = t4.reshape(TILES * 4, CPT // 128, 128)
    row_sc2 = row_p.reshape(TILES * 4, CPT // 4 // 128, 128)
    acc = _sc_scatter(t_r, row_sc2)

    acc_r = acc.reshape(TILES, 8, NP * 4 // 8)
    part_r = part.reshape(8, NP * 4 // 8)
    out = _final_call(acc_r, part_r)

    return out.reshape(NP, 4)[:N, :3]
